# baseline (speedup 1.0000x reference)
import functools

import jax
import jax.numpy as jnp
from jax import lax
from jax.experimental import pallas as pl
from jax.experimental.pallas import tpu as pltpu
from jax.experimental.pallas import tpu_sc as plsc

N = 2048
K = 5
BR = 256
NI = N // BR

_NW = 32
_RPW = N // _NW
_CH = 16
_NCHUNK = _RPW // _CH
_IL = 4
_LANES = 16


def _score_kernel(x_ref, y_ref, s_ref, y2_ref):
    x = x_ref[...]
    y = y_ref[...]
    g = jax.lax.dot_general(
        x, y, (((1,), (1,)), ((), ())), preferred_element_type=jnp.float32
    )
    xsq = x * x
    x2 = jnp.sum(xsq, axis=1, keepdims=True)
    s_ref[...] = x2 - 2.0 * g
    ones = jnp.ones((8, N), jnp.float32)
    y2_ref[...] = jax.lax.dot_general(
        ones, xsq, (((1,), (1,)), ((), ())),
        preferred_element_type=jnp.float32,
    )


def _sc_insert(regs, v):
    out = list(regs)
    for t in range(K - 1, -1, -1):
        lo = jnp.minimum(out[t], v)
        v = jnp.maximum(out[t], v)
        out[t] = lo
    return out


def _sc_kernel(s_hbm, y2_hbm, out_hbm, y2_v, buf0, buf1, outb, sem0, sem1):
    cid = lax.axis_index("c")
    sid = lax.axis_index("s")
    wid = sid * 2 + cid
    base = wid * _RPW

    pltpu.sync_copy(y2_hbm, y2_v)

    inf16 = jnp.full((_LANES,), jnp.inf, jnp.float32)

    def process_chunk(buf, k):
        def group_body(gr, carry):
            def cc_body(cc, regs):
                off = cc * _LANES
                y2c = y2_v[pl.ds(off, _LANES)]
                new = []
                for q in range(_IL):
                    v = buf[_IL * gr + q, pl.ds(off, _LANES)] + y2c
                    new.extend(_sc_insert(regs[q * K:(q + 1) * K], v))
                return new
            regs = lax.fori_loop(0, N // _LANES, cc_body,
                                 [inf16] * (K * _IL))
            for q in range(_IL):
                r = k * _CH + _IL * gr + q
                for u in range(K):
                    outb[r, pl.ds(u * _LANES, _LANES)] = regs[q * K + u]
            return carry
        lax.fori_loop(0, _CH // _IL, group_body, 0)

    bufs = [buf0, buf1]
    sems = [sem0, sem1]
    cp = pltpu.async_copy(s_hbm.at[pl.ds(base, _CH)], buf0, sem0)
    for k in range(_NCHUNK):
        nxt = None
        if k + 1 < _NCHUNK:
            nxt = pltpu.async_copy(
                s_hbm.at[pl.ds(base + (k + 1) * _CH, _CH)],
                bufs[(k + 1) % 2], sems[(k + 1) % 2])
        cp.wait()
        process_chunk(bufs[k % 2], k)
        cp = nxt

    pltpu.sync_copy(outb, out_hbm.at[pl.ds(base, _RPW)])


_CAND = K * _LANES


def _std_kernel(v_ref, out_ref):
    cand = v_ref[...]
    iota = jax.lax.broadcasted_iota(jnp.int32, (N, _CAND), 1)
    sels = []
    for t in range(K):
        m = jnp.min(cand, axis=1, keepdims=True)
        c0 = jnp.min(jnp.where(cand == m, iota, _CAND), axis=1,
                     keepdims=True)
        cand = jnp.where(iota == c0, jnp.inf, cand)
        sels.append(m)
    d2 = jnp.maximum(jnp.concatenate(sels, axis=1), 0.0)
    d = jnp.where(d2 > 0.0, jnp.sqrt(jnp.where(d2 > 0.0, d2, 1.0)), 0.0)
    n = jnp.float32(N * K)
    mean = jnp.sum(d) / n
    dev = d - mean
    out_ref[...] = jnp.full(
        (1, 1), -jnp.sqrt(jnp.sum(dev * dev) / (n - 1.0)), jnp.float32
    )


def kernel(latent):
    sample = latent[0]

    s_full, y2x8 = pl.pallas_call(
        _score_kernel,
        grid=(NI,),
        in_specs=[
            pl.BlockSpec((BR, N), lambda i: (i, 0)),
            pl.BlockSpec((N, N), lambda i: (0, 0)),
        ],
        out_specs=[
            pl.BlockSpec((BR, N), lambda i: (i, 0)),
            pl.BlockSpec((8, BR), lambda i: (0, i)),
        ],
        out_shape=[
            jax.ShapeDtypeStruct((N, N), jnp.float32),
            jax.ShapeDtypeStruct((8, N), jnp.float32),
        ],
    )(sample, sample)

    y2vec = y2x8[0]

    mesh = plsc.VectorSubcoreMesh(core_axis_name="c", subcore_axis_name="s")
    sc_topk = functools.partial(
        pl.kernel,
        mesh=mesh,
        out_type=jax.ShapeDtypeStruct((N, _CAND), jnp.float32),
        scratch_types=[
            pltpu.VMEM((N,), jnp.float32),
            pltpu.VMEM((_CH, N), jnp.float32),
            pltpu.VMEM((_CH, N), jnp.float32),
            pltpu.VMEM((_RPW, _CAND), jnp.float32),
            pltpu.SemaphoreType.DMA,
            pltpu.SemaphoreType.DMA,
        ],
    )(_sc_kernel)
    knn2 = sc_topk(s_full, y2vec)

    out = pl.pallas_call(
        _std_kernel,
        in_specs=[pl.BlockSpec((N, _CAND), lambda: (0, 0))],
        out_specs=pl.BlockSpec((1, 1), lambda: (0, 0)),
        out_shape=jax.ShapeDtypeStruct((1, 1), jnp.float32),
    )(knn2)
    return out[0, 0]


# speedup vs baseline: 1.6678x; 1.6678x over previous
import jax
import jax.numpy as jnp
from jax.experimental import pallas as pl
from jax.experimental.pallas import tpu as pltpu

N = 2048
K = 5
BR = 256
NI = N // BR
_PADR = 8


def _knn_std_kernel(y_ref, out_ref, y2_ref, acc_ref):
    i = pl.program_id(0)

    y = y_ref[...]
    x = y_ref[pl.ds(pl.multiple_of(i * BR, BR), BR), :]

    g = jax.lax.dot_general(
        y, x, (((1,), (1,)), ((), ())), preferred_element_type=jnp.float32
    )

    @pl.when(i == 0)
    def _():
        y2_ref[...] = jnp.sum(y * y, axis=1, keepdims=True)

    y2 = y2_ref[...]
    st = y2 - 2.0 * g

    inf = jnp.full((_PADR, BR), jnp.inf, jnp.float32)
    s = [inf] * K
    for r in range(N // _PADR):
        v = st[r * _PADR:(r + 1) * _PADR, :]
        for t in range(K - 1, -1, -1):
            lo = jnp.minimum(s[t], v)
            v = jnp.maximum(s[t], v)
            s[t] = lo

    cand = jnp.concatenate(s, axis=0)
    H = K * _PADR
    iota = jax.lax.broadcasted_iota(jnp.int32, (H, BR), 0)
    row = jax.lax.broadcasted_iota(jnp.int32, (_PADR, BR), 0)
    sel = jnp.full((_PADR, BR), jnp.inf, jnp.float32)
    for t in range(K):
        m = jnp.min(cand, axis=0, keepdims=True)
        r0 = jnp.min(jnp.where(cand == m, iota, H), axis=0, keepdims=True)
        cand = jnp.where(iota == r0, jnp.inf, cand)
        sel = jnp.where(row == t, m, sel)

    ones = jnp.ones((8, N), jnp.float32)
    x2 = jax.lax.dot_general(
        ones, x * x, (((1,), (1,)), ((), ())),
        preferred_element_type=jnp.float32,
    )[0:1, :]
    d2 = jnp.maximum(x2 + sel, 0.0)
    knn = jnp.where(d2 > 0.0, jnp.sqrt(jnp.where(d2 > 0.0, d2, 1.0)), 0.0)
    valid = row < K
    knn = jnp.where(valid, knn, 0.0)
    nb = jnp.float32(BR * K)
    mean_b = jnp.sum(knn) / nb
    dev = jnp.where(valid, knn - mean_b, 0.0)
    m2_b = jnp.sum(dev * dev)

    @pl.when(i == 0)
    def _():
        acc_ref[0] = nb
        acc_ref[1] = mean_b
        acc_ref[2] = m2_b

    @pl.when(i > 0)
    def _():
        na = acc_ref[0]
        mean_a = acc_ref[1]
        m2_a = acc_ref[2]
        n = na + nb
        delta = mean_b - mean_a
        acc_ref[0] = n
        acc_ref[1] = mean_a + delta * (nb / n)
        acc_ref[2] = m2_a + m2_b + delta * delta * (na * nb / n)

    @pl.when(i == NI - 1)
    def _():
        n = acc_ref[0]
        out_ref[...] = jnp.full(
            (1, 1), -jnp.sqrt(acc_ref[2] / (n - 1.0)), jnp.float32
        )


def kernel(latent):
    sample = latent[0]
    out = pl.pallas_call(
        _knn_std_kernel,
        grid=(NI,),
        in_specs=[
            pl.BlockSpec((N, N), lambda i: (0, 0)),
        ],
        out_specs=pl.BlockSpec((1, 1), lambda i: (0, 0)),
        out_shape=jax.ShapeDtypeStruct((1, 1), jnp.float32),
        scratch_shapes=[
            pltpu.VMEM((N, 1), jnp.float32),
            pltpu.SMEM((4,), jnp.float32),
        ],
    )(sample)
    return out[0, 0]


# speedup vs baseline: 1.8788x; 1.1265x over previous
import jax
import jax.numpy as jnp
from jax.experimental import pallas as pl
from jax.experimental.pallas import tpu as pltpu

N = 2048
K = 5
BR = 512
NI = N // BR
_PADR = 8


def _knn_std_kernel(y_ref, out_ref, y2_ref, acc_ref):
    i = pl.program_id(0)

    y = y_ref[...]
    x = y_ref[pl.ds(pl.multiple_of(i * BR, BR), BR), :]

    g = jax.lax.dot_general(
        y, x, (((1,), (1,)), ((), ())), preferred_element_type=jnp.float32
    )

    @pl.when(i == 0)
    def _():
        y2_ref[...] = jnp.sum(y * y, axis=1, keepdims=True)

    y2 = y2_ref[...]
    st = y2 - 2.0 * g

    inf = jnp.full((_PADR, BR), jnp.inf, jnp.float32)
    s = [inf] * K
    for r in range(N // _PADR):
        v = st[r * _PADR:(r + 1) * _PADR, :]
        for t in range(K - 1, -1, -1):
            lo = jnp.minimum(s[t], v)
            v = jnp.maximum(s[t], v)
            s[t] = lo

    cand = jnp.concatenate(s, axis=0)
    H = K * _PADR
    iota = jax.lax.broadcasted_iota(jnp.int32, (H, BR), 0)
    row = jax.lax.broadcasted_iota(jnp.int32, (_PADR, BR), 0)
    sel = jnp.full((_PADR, BR), jnp.inf, jnp.float32)
    for t in range(K):
        m = jnp.min(cand, axis=0, keepdims=True)
        r0 = jnp.min(jnp.where(cand == m, iota, H), axis=0, keepdims=True)
        cand = jnp.where(iota == r0, jnp.inf, cand)
        sel = jnp.where(row == t, m, sel)

    ones = jnp.ones((8, N), jnp.float32)
    x2 = jax.lax.dot_general(
        ones, x * x, (((1,), (1,)), ((), ())),
        preferred_element_type=jnp.float32,
    )[0:1, :]
    d2 = jnp.maximum(x2 + sel, 0.0)
    knn = jnp.where(d2 > 0.0, jnp.sqrt(jnp.where(d2 > 0.0, d2, 1.0)), 0.0)
    valid = row < K
    knn = jnp.where(valid, knn, 0.0)
    nb = jnp.float32(BR * K)
    mean_b = jnp.sum(knn) / nb
    dev = jnp.where(valid, knn - mean_b, 0.0)
    m2_b = jnp.sum(dev * dev)

    @pl.when(i == 0)
    def _():
        acc_ref[0] = nb
        acc_ref[1] = mean_b
        acc_ref[2] = m2_b

    @pl.when(i > 0)
    def _():
        na = acc_ref[0]
        mean_a = acc_ref[1]
        m2_a = acc_ref[2]
        n = na + nb
        delta = mean_b - mean_a
        acc_ref[0] = n
        acc_ref[1] = mean_a + delta * (nb / n)
        acc_ref[2] = m2_a + m2_b + delta * delta * (na * nb / n)

    @pl.when(i == NI - 1)
    def _():
        n = acc_ref[0]
        out_ref[...] = jnp.full(
            (1, 1), -jnp.sqrt(acc_ref[2] / (n - 1.0)), jnp.float32
        )


def kernel(latent):
    sample = latent[0]
    out = pl.pallas_call(
        _knn_std_kernel,
        grid=(NI,),
        in_specs=[
            pl.BlockSpec((N, N), lambda i: (0, 0)),
        ],
        out_specs=pl.BlockSpec((1, 1), lambda i: (0, 0)),
        out_shape=jax.ShapeDtypeStruct((1, 1), jnp.float32),
        scratch_shapes=[
            pltpu.VMEM((N, 1), jnp.float32),
            pltpu.SMEM((4,), jnp.float32),
        ],
    )(sample)
    return out[0, 0]


# speedup vs baseline: 1.9155x; 1.0195x over previous
import jax
import jax.numpy as jnp
from jax.experimental import pallas as pl
from jax.experimental.pallas import tpu as pltpu

N = 2048
K = 5
BR = 1024
NI = N // BR
_PADR = 8


def _knn_std_kernel(y_ref, out_ref, y2_ref, acc_ref):
    i = pl.program_id(0)

    y = y_ref[...]
    x = y_ref[pl.ds(pl.multiple_of(i * BR, BR), BR), :]

    g = jax.lax.dot_general(
        y, x, (((1,), (1,)), ((), ())), preferred_element_type=jnp.float32
    )

    @pl.when(i == 0)
    def _():
        y2_ref[...] = jnp.sum(y * y, axis=1, keepdims=True)

    y2 = y2_ref[...]
    st = y2 - 2.0 * g

    inf = jnp.full((_PADR, BR), jnp.inf, jnp.float32)
    s = [inf] * K
    for r in range(N // _PADR):
        v = st[r * _PADR:(r + 1) * _PADR, :]
        for t in range(K - 1, -1, -1):
            lo = jnp.minimum(s[t], v)
            v = jnp.maximum(s[t], v)
            s[t] = lo

    cand = jnp.concatenate(s, axis=0)
    H = K * _PADR
    iota = jax.lax.broadcasted_iota(jnp.int32, (H, BR), 0)
    row = jax.lax.broadcasted_iota(jnp.int32, (_PADR, BR), 0)
    sel = jnp.full((_PADR, BR), jnp.inf, jnp.float32)
    for t in range(K):
        m = jnp.min(cand, axis=0, keepdims=True)
        r0 = jnp.min(jnp.where(cand == m, iota, H), axis=0, keepdims=True)
        cand = jnp.where(iota == r0, jnp.inf, cand)
        sel = jnp.where(row == t, m, sel)

    ones = jnp.ones((8, N), jnp.float32)
    x2 = jax.lax.dot_general(
        ones, x * x, (((1,), (1,)), ((), ())),
        preferred_element_type=jnp.float32,
    )[0:1, :]
    d2 = jnp.maximum(x2 + sel, 0.0)
    knn = jnp.where(d2 > 0.0, jnp.sqrt(jnp.where(d2 > 0.0, d2, 1.0)), 0.0)
    valid = row < K
    knn = jnp.where(valid, knn, 0.0)
    nb = jnp.float32(BR * K)
    mean_b = jnp.sum(knn) / nb
    dev = jnp.where(valid, knn - mean_b, 0.0)
    m2_b = jnp.sum(dev * dev)

    @pl.when(i == 0)
    def _():
        acc_ref[0] = nb
        acc_ref[1] = mean_b
        acc_ref[2] = m2_b

    @pl.when(i > 0)
    def _():
        na = acc_ref[0]
        mean_a = acc_ref[1]
        m2_a = acc_ref[2]
        n = na + nb
        delta = mean_b - mean_a
        acc_ref[0] = n
        acc_ref[1] = mean_a + delta * (nb / n)
        acc_ref[2] = m2_a + m2_b + delta * delta * (na * nb / n)

    @pl.when(i == NI - 1)
    def _():
        n = acc_ref[0]
        out_ref[...] = jnp.full(
            (1, 1), -jnp.sqrt(acc_ref[2] / (n - 1.0)), jnp.float32
        )


def kernel(latent):
    sample = latent[0]
    out = pl.pallas_call(
        _knn_std_kernel,
        grid=(NI,),
        in_specs=[
            pl.BlockSpec((N, N), lambda i: (0, 0)),
        ],
        out_specs=pl.BlockSpec((1, 1), lambda i: (0, 0)),
        out_shape=jax.ShapeDtypeStruct((1, 1), jnp.float32),
        scratch_shapes=[
            pltpu.VMEM((N, 1), jnp.float32),
            pltpu.SMEM((4,), jnp.float32),
        ],
    )(sample)
    return out[0, 0]
